# Initial kernel scaffold; baseline (speedup 1.0000x reference)
import jax
import jax.numpy as jnp
from jax.experimental import pallas as pl


def kernel(indices, weight):
    raise NotImplementedError("write your pallas kernel here")


# speedup vs baseline: 1.6843x; 1.6843x over previous
import functools

import jax
import jax.numpy as jnp
from jax import lax
from jax.experimental import pallas as pl
from jax.experimental.pallas import tpu as pltpu
from jax.experimental.pallas import tpu_sc as plsc

NUM_NODES = 1000000
EMBED_DIM = 64

_B = 16384 * 50
_C = 128
_NCHUNK = _B // _C
_NW = 32
_CPW = _NCHUNK // _NW


def _make_gather():
    mesh = plsc.VectorSubcoreMesh(core_axis_name="c", subcore_axis_name="s")

    @functools.partial(
        pl.kernel,
        out_type=jax.ShapeDtypeStruct((_B, EMBED_DIM), jnp.float32),
        mesh=mesh,
        scratch_types=[
            pltpu.VMEM((_CPW, _C), jnp.int32),
            pltpu.VMEM((_C, EMBED_DIM), jnp.float32),
            pltpu.SemaphoreType.DMA,
        ],
        compiler_params=pltpu.CompilerParams(use_tc_tiling_on_sc=False),
    )
    def gather_kernel(idx_hbm, table_hbm, out_hbm, idx_v, rows_v, sem):
        wid = lax.axis_index("s") * 2 + lax.axis_index("c")
        base = wid * _CPW
        pltpu.sync_copy(idx_hbm.at[pl.ds(base, _CPW)], idx_v)

        def body(j, carry):
            pltpu.async_copy(table_hbm.at[idx_v.at[j]], rows_v, sem).wait()
            pltpu.sync_copy(
                rows_v, out_hbm.at[pl.ds((base + j) * _C, _C)])
            return carry

        lax.fori_loop(0, _CPW, body, 0, unroll=False)

    return gather_kernel


_gather = _make_gather()


def kernel(indices, weight):
    idx2d = indices.astype(jnp.int32).reshape(_NCHUNK, _C)
    out = _gather(idx2d, weight)
    return out.reshape(indices.shape[0], indices.shape[1], EMBED_DIM)


# speedup vs baseline: 1.8692x; 1.1098x over previous
import functools

import jax
import jax.numpy as jnp
from jax import lax
from jax.experimental import pallas as pl
from jax.experimental.pallas import tpu as pltpu
from jax.experimental.pallas import tpu_sc as plsc

NUM_NODES = 1000000
EMBED_DIM = 64

_B = 16384 * 50
_C = 128
_NCHUNK = _B // _C
_NW = 32
_CPW = _NCHUNK // _NW
_G = 2
_NB = 4
_RPB = _G * _C
_T = _CPW // (_G * _NB)


def _make_gather():
    mesh = plsc.VectorSubcoreMesh(core_axis_name="c", subcore_axis_name="s")

    @functools.partial(
        pl.kernel,
        out_type=jax.ShapeDtypeStruct((_B, EMBED_DIM), jnp.float32),
        mesh=mesh,
        scratch_types=(
            [pltpu.VMEM((_CPW, _C), jnp.int32)]
            + [pltpu.VMEM((_RPB, EMBED_DIM), jnp.float32)] * _NB
            + [pltpu.SemaphoreType.DMA] * (2 * _NB)
        ),
        compiler_params=pltpu.CompilerParams(use_tc_tiling_on_sc=False),
    )
    def gather_kernel(idx_hbm, table_hbm, out_hbm, idx_v, *bufs_and_sems):
        bufs = bufs_and_sems[:_NB]
        gsem = bufs_and_sems[_NB:2 * _NB]
        ssem = bufs_and_sems[2 * _NB:]

        wid = lax.axis_index("s") * 2 + lax.axis_index("c")
        chunk_base = wid * _CPW
        row_base = chunk_base * _C
        pltpu.sync_copy(idx_hbm.at[pl.ds(chunk_base, _CPW)], idx_v)

        def issue_gathers(blk, b):
            for k in range(_G):
                pltpu.async_copy(
                    table_hbm.at[idx_v.at[blk * _G + k]],
                    bufs[b].at[pl.ds(k * _C, _C)],
                    gsem[b])

        for b in range(_NB):
            issue_gathers(b, b)

        def body(t, carry):
            for b in range(_NB):
                blk = t * _NB + b
                pltpu.make_async_copy(
                    table_hbm.at[pl.ds(0, _RPB)], bufs[b], gsem[b]).wait()
                pltpu.async_copy(
                    bufs[b],
                    out_hbm.at[pl.ds(row_base + blk * _RPB, _RPB)],
                    ssem[b])
            for b in range(_NB):
                pltpu.make_async_copy(
                    bufs[b], out_hbm.at[pl.ds(0, _RPB)], ssem[b]).wait()
                pl.when(t != _T - 1)(
                    functools.partial(issue_gathers, (t + 1) * _NB + b, b))
            return carry

        lax.fori_loop(0, _T, body, 0, unroll=False)

    return gather_kernel


_gather = _make_gather()


def kernel(indices, weight):
    idx2d = indices.astype(jnp.int32).reshape(_NCHUNK, _C)
    out = _gather(idx2d, weight)
    return out.reshape(indices.shape[0], indices.shape[1], EMBED_DIM)
